# baseline (speedup 1.0000x reference)
import functools

import jax
import jax.numpy as jnp
from jax import lax
from jax.experimental import pallas as pl
from jax.experimental.pallas import tpu as pltpu
from jax.experimental.pallas import tpu_sc as plsc

B = 4096
L = 200
D = 128
N = B * L

_info = plsc.get_sparse_core_info()
_NC = _info.num_cores
_NS = _info.num_subcores
_NW = _NC * _NS
_ROWS = N // _NW
_CHUNK = 256
_NCHUNK = _ROWS // _CHUNK

_mesh = plsc.VectorSubcoreMesh(core_axis_name="c", subcore_axis_name="s")


@functools.partial(
    pl.kernel,
    mesh=_mesh,
    out_type=jax.ShapeDtypeStruct((N, D), jnp.float32),
    scratch_types=[
        pltpu.VMEM((_CHUNK,), jnp.int32),
        pltpu.VMEM((_CHUNK, D), jnp.float32),
        pltpu.SemaphoreType.DMA,
    ],
)
def _sc_gather(x_hbm, table_hbm, out_hbm, idx_v, rows_v, sem):
    wid = lax.axis_index("s") * _NC + lax.axis_index("c")
    base = wid * _ROWS

    def step(i, carry):
        off = base + i * _CHUNK
        pltpu.sync_copy(x_hbm.at[pl.ds(off, _CHUNK)], idx_v)
        pltpu.async_copy(table_hbm.at[idx_v], rows_v, sem).wait()
        pltpu.sync_copy(rows_v, out_hbm.at[pl.ds(off, _CHUNK)])
        return carry

    lax.fori_loop(0, _NCHUNK, step, 0)


def _tc_body(x_ref, emb_ref, fmap_ref, len_ref):
    e = emb_ref[0]
    fmap_ref[0] = e.T
    xr = x_ref[0]
    len_ref[0] = jnp.sum((xr != 0).astype(jnp.int32)).reshape(1, 1)


_transpose_len = pl.pallas_call(
    _tc_body,
    grid=(B,),
    in_specs=[
        pl.BlockSpec((1, 1, L), lambda i: (i, 0, 0)),
        pl.BlockSpec((1, L, D), lambda i: (i, 0, 0)),
    ],
    out_specs=[
        pl.BlockSpec((1, D, L), lambda i: (i, 0, 0)),
        pl.BlockSpec((1, 1, 1), lambda i: (i, 0, 0)),
    ],
    out_shape=[
        jax.ShapeDtypeStruct((B, D, L), jnp.float32),
        jax.ShapeDtypeStruct((B, 1, 1), jnp.int32),
    ],
)


def kernel(x, table):
    x = x.astype(jnp.int32)
    emb = _sc_gather(x.reshape(N), table)
    fmap, lens = _transpose_len(x.reshape(B, 1, L), emb.reshape(B, L, D))
    return fmap, lens.reshape(B)


# speedup vs baseline: 2.5171x; 2.5171x over previous
import functools

import jax
import jax.numpy as jnp
from jax import lax
from jax.experimental import pallas as pl
from jax.experimental.pallas import tpu as pltpu
from jax.experimental.pallas import tpu_sc as plsc

B = 4096
L = 200
D = 128
N = B * L

_info = plsc.get_sparse_core_info()
_NC = _info.num_cores
_NS = _info.num_subcores
_NW = _NC * _NS
_ROWS = N // _NW
_CHUNK = 256
_NCHUNK = _ROWS // _CHUNK

_mesh = plsc.VectorSubcoreMesh(core_axis_name="c", subcore_axis_name="s")


@functools.partial(
    pl.kernel,
    mesh=_mesh,
    out_type=jax.ShapeDtypeStruct((N, D), jnp.float32),
    scratch_types=[
        pltpu.VMEM((_CHUNK,), jnp.int32),
        pltpu.VMEM((_CHUNK, D), jnp.float32),
        pltpu.SemaphoreType.DMA,
    ],
)
def _sc_gather(x_hbm, table_hbm, out_hbm, idx_v, rows_v, sem):
    wid = lax.axis_index("s") * _NC + lax.axis_index("c")
    base = wid * _ROWS

    def step(i, carry):
        off = base + i * _CHUNK
        pltpu.sync_copy(x_hbm.at[pl.ds(off, _CHUNK)], idx_v)
        pltpu.async_copy(table_hbm.at[idx_v], rows_v, sem).wait()
        pltpu.sync_copy(rows_v, out_hbm.at[pl.ds(off, _CHUNK)])
        return carry

    lax.fori_loop(0, _NCHUNK, step, 0)


_BT = 16


def _tc_body(x_ref, emb_ref, fmap_ref, len_ref):
    e = emb_ref[...]
    fmap_ref[...] = jnp.transpose(e, (0, 2, 1))
    xr = x_ref[...]
    len_ref[...] = jnp.sum((xr != 0).astype(jnp.int32), axis=2, keepdims=True)


_transpose_len = pl.pallas_call(
    _tc_body,
    grid=(B // _BT,),
    in_specs=[
        pl.BlockSpec((_BT, 1, L), lambda i: (i, 0, 0)),
        pl.BlockSpec((_BT, L, D), lambda i: (i, 0, 0)),
    ],
    out_specs=[
        pl.BlockSpec((_BT, D, L), lambda i: (i, 0, 0)),
        pl.BlockSpec((_BT, 1, 1), lambda i: (i, 0, 0)),
    ],
    out_shape=[
        jax.ShapeDtypeStruct((B, D, L), jnp.float32),
        jax.ShapeDtypeStruct((B, 1, 1), jnp.int32),
    ],
)


def kernel(x, table):
    x = x.astype(jnp.int32)
    emb = _sc_gather(x.reshape(N), table)
    fmap, lens = _transpose_len(x.reshape(B, 1, L), emb.reshape(B, L, D))
    return fmap, lens.reshape(B)


# speedup vs baseline: 2.6762x; 1.0632x over previous
import functools

import jax
import jax.numpy as jnp
from jax import lax
from jax.experimental import pallas as pl
from jax.experimental.pallas import tpu as pltpu
from jax.experimental.pallas import tpu_sc as plsc

B = 4096
L = 200
D = 128
N = B * L

_info = plsc.get_sparse_core_info()
_NC = _info.num_cores
_NS = _info.num_subcores
_NW = _NC * _NS
_ROWS = N // _NW
_CHUNK = 256
_NCHUNK = _ROWS // _CHUNK

_mesh = plsc.VectorSubcoreMesh(core_axis_name="c", subcore_axis_name="s")


@functools.partial(
    pl.kernel,
    mesh=_mesh,
    out_type=jax.ShapeDtypeStruct((N, D), jnp.float32),
    scratch_types=[
        pltpu.VMEM((_CHUNK,), jnp.int32),
        pltpu.VMEM((_CHUNK, D), jnp.float32),
        pltpu.SemaphoreType.DMA,
    ],
)
def _sc_gather(x_hbm, table_hbm, out_hbm, idx_v, rows_v, sem):
    wid = lax.axis_index("s") * _NC + lax.axis_index("c")
    base = wid * _ROWS

    def step(i, carry):
        off = base + i * _CHUNK
        pltpu.sync_copy(x_hbm.at[pl.ds(off, _CHUNK)], idx_v)
        pltpu.async_copy(table_hbm.at[idx_v], rows_v, sem).wait()
        pltpu.sync_copy(rows_v, out_hbm.at[pl.ds(off, _CHUNK)])
        return carry

    lax.fori_loop(0, _NCHUNK, step, 0)


_BT = 64


def _tc_body(x_ref, emb_ref, fmap_ref, len_ref):
    e = emb_ref[...]
    fmap_ref[...] = jnp.transpose(e, (0, 2, 1))
    xr = x_ref[...]
    len_ref[...] = jnp.sum((xr != 0).astype(jnp.int32), axis=2, keepdims=True)


_transpose_len = pl.pallas_call(
    _tc_body,
    grid=(B // _BT,),
    in_specs=[
        pl.BlockSpec((_BT, 1, L), lambda i: (i, 0, 0)),
        pl.BlockSpec((_BT, L, D), lambda i: (i, 0, 0)),
    ],
    out_specs=[
        pl.BlockSpec((_BT, D, L), lambda i: (i, 0, 0)),
        pl.BlockSpec((_BT, 1, 1), lambda i: (i, 0, 0)),
    ],
    out_shape=[
        jax.ShapeDtypeStruct((B, D, L), jnp.float32),
        jax.ShapeDtypeStruct((B, 1, 1), jnp.int32),
    ],
)


def kernel(x, table):
    x = x.astype(jnp.int32)
    emb = _sc_gather(x.reshape(N), table)
    fmap, lens = _transpose_len(x.reshape(B, 1, L), emb.reshape(B, L, D))
    return fmap, lens.reshape(B)


# speedup vs baseline: 2.8290x; 1.0571x over previous
import functools

import jax
import jax.numpy as jnp
from jax import lax
from jax.experimental import pallas as pl
from jax.experimental.pallas import tpu as pltpu
from jax.experimental.pallas import tpu_sc as plsc

B = 4096
L = 200
D = 128
N = B * L

_info = plsc.get_sparse_core_info()
_NC = _info.num_cores
_NS = _info.num_subcores
_NW = _NC * _NS
_ROWS = N // _NW
_CHUNK = 256
_NCHUNK = _ROWS // _CHUNK

_mesh = plsc.VectorSubcoreMesh(core_axis_name="c", subcore_axis_name="s")


@functools.partial(
    pl.kernel,
    mesh=_mesh,
    out_type=jax.ShapeDtypeStruct((N, D), jnp.float32),
    scratch_types=[
        pltpu.VMEM((_CHUNK,), jnp.int32),
        pltpu.VMEM((_CHUNK, D), jnp.float32),
        pltpu.SemaphoreType.DMA,
    ],
)
def _sc_gather(x_hbm, table_hbm, out_hbm, idx_v, rows_v, sem):
    wid = lax.axis_index("s") * _NC + lax.axis_index("c")
    base = wid * _ROWS

    def step(i, carry):
        off = base + i * _CHUNK
        pltpu.sync_copy(x_hbm.at[pl.ds(off, _CHUNK)], idx_v)
        pltpu.async_copy(table_hbm.at[idx_v], rows_v, sem).wait()
        pltpu.sync_copy(rows_v, out_hbm.at[pl.ds(off, _CHUNK)])
        return carry

    lax.fori_loop(0, _NCHUNK, step, 0)


_BT = 128


def _tc_body(emb_ref, fmap_ref):
    e = emb_ref[...]
    fmap_ref[...] = jnp.transpose(e, (0, 2, 1))


_transpose = pl.pallas_call(
    _tc_body,
    grid=(B // _BT,),
    in_specs=[pl.BlockSpec((_BT, L, D), lambda i: (i, 0, 0))],
    out_specs=pl.BlockSpec((_BT, D, L), lambda i: (i, 0, 0)),
    out_shape=jax.ShapeDtypeStruct((B, D, L), jnp.float32),
)


def _len_body(x_ref, len_ref):
    xr = x_ref[...]
    len_ref[...] = jnp.sum((xr != 0).astype(jnp.int32), axis=2, keepdims=True)


_lengths = pl.pallas_call(
    _len_body,
    grid=(1,),
    in_specs=[pl.BlockSpec((B, 1, L), lambda i: (0, 0, 0))],
    out_specs=pl.BlockSpec((B, 1, 1), lambda i: (0, 0, 0)),
    out_shape=jax.ShapeDtypeStruct((B, 1, 1), jnp.int32),
)


def kernel(x, table):
    x = x.astype(jnp.int32)
    emb = _sc_gather(x.reshape(N), table)
    fmap = _transpose(emb.reshape(B, L, D))
    lens = _lengths(x.reshape(B, 1, L))
    return fmap, lens.reshape(B)
